# baseline (speedup 1.0000x reference)
import functools

import jax
import jax.numpy as jnp
from jax.experimental import pallas as pl
from jax.experimental.pallas import tpu as pltpu

_D_VALS = (1.5, 0.75, 0.3725, 0.18625, 0.093125)
_BN_EPS = 1e-5
_LANE = 128
_H = 32
_W = 32
_HP = _H + 2
_WP = _W + 2
_P = _HP * _WP
_M = 40
_VMEM_LIMIT = 64 * 1024 * 1024


def _fs_code(act, cnt):
    c = act
    for d in _D_VALS:
        fire = c > d
        c = jnp.where(fire, c - d, c)
        cnt = cnt + jnp.where(fire, jnp.float32(1.0), jnp.float32(0.0))
    return act - c, cnt


def _to_padded_slab(x_img):
    x_wide = jnp.concatenate(
        [x_img, jnp.zeros((_LANE - x_img.shape[0], _H * _W), jnp.float32)],
        axis=0)
    xt = jnp.transpose(x_wide, (1, 0))
    z_ring = jnp.zeros((_WP + 1, _LANE), jnp.float32)
    z_edge = jnp.zeros((2, _LANE), jnp.float32)
    pieces = [z_ring]
    for r in range(_H):
        pieces.append(xt[r * _W:(r + 1) * _W])
        pieces.append(z_edge if r < _H - 1 else z_ring)
    return jnp.concatenate(pieces, axis=0)


def _interior_t(slab):
    rows = [slab[34 * r + 35:34 * r + 67] for r in range(_H)]
    return jnp.transpose(jnp.concatenate(rows, axis=0), (1, 0))


def _block_kernel(x_ref, mask_ref, sc1_ref, sh1_ref, w1_ref, sc2_ref,
                  sh2_ref, w2_ref, out_ref, spk_ref, zbuf0, zbuf1):
    mask = mask_ref[...]
    zbufs = (zbuf0, zbuf1)
    slab0 = [_to_padded_slab(x_ref[0]), _to_padded_slab(x_ref[1])]
    slab1 = [jnp.zeros((_P, _LANE), jnp.float32) for _ in range(2)]
    cnt = [jnp.zeros((_P, _LANE), jnp.float32) for _ in range(2)]

    for zb in zbufs:
        zb[pl.ds(0, _M), :] = jnp.zeros((_M, _LANE), jnp.float32)
        zb[pl.ds(_M + _P, _M), :] = jnp.zeros((_M, _LANE), jnp.float32)

    for l in range(4):
        y = [None, None]
        for g in range(2):
            act = jnp.maximum(slab0[g] * sc1_ref[l] + sh1_ref[l], 0.0) * mask
            zw, cnt[g] = _fs_code(act, cnt[g])
            y[g] = jnp.dot(zw, w1_ref[l],
                           preferred_element_type=jnp.float32)
            if l == 3:
                act_b = jnp.maximum(slab1[g] * sc1_ref[4] + sh1_ref[4],
                                    0.0) * mask
                zw_b, cnt[g] = _fs_code(act_b, cnt[g])
                y[g] = y[g] + jnp.dot(zw_b, w1_ref[4],
                                      preferred_element_type=jnp.float32)

        for g in range(2):
            act2 = jnp.maximum(y[g] * sc2_ref[l] + sh2_ref[l], 0.0) * mask
            zw2, cnt[g] = _fs_code(act2, cnt[g])
            zbufs[g][pl.ds(_M, _P), :] = zw2
            acc = jnp.zeros((_P, _LANE), jnp.float32)
            for t in range(9):
                ky, kx = t // 3, t % 3
                off = _M + (ky - 1) * _WP + (kx - 1)
                acc = acc + jnp.dot(zbufs[g][pl.ds(off, _P), :],
                                    w2_ref[9 * l + t],
                                    preferred_element_type=jnp.float32)
            if l < 2:
                slab0[g] = slab0[g] + acc
            else:
                slab1[g] = slab1[g] + acc

    for g in range(2):
        t0 = _interior_t(slab0[g])
        t1 = _interior_t(slab1[g])
        out_ref[g, 0:_LANE, :] = t0
        out_ref[g, _LANE:_LANE + 64, :] = t1[0:64, :]
        spk_ref[g] = jnp.sum(cnt[g], axis=0, keepdims=True)


def _bn_fold(bn):
    gamma, beta, mean, var = bn[0], bn[1], bn[2], bn[3]
    scale = gamma / jnp.sqrt(var + _BN_EPS)
    return scale, beta - mean * scale


def _pad_lanes(v, width):
    return jnp.pad(v, (0, width - v.shape[0])).reshape(1, width)


@functools.partial(jax.jit, static_argnames=())
def _forward(x, bn1s, w1s, bn2s, w2s):
    b, c_in = x.shape[0], x.shape[1]
    growth = w2s[0].shape[0]
    c_mid = w2s[0].shape[1]

    x_in = x.reshape(b, c_in, _H * _W)

    hh = jnp.arange(_HP).reshape(_HP, 1)
    ww = jnp.arange(_WP).reshape(1, _WP)
    mask = ((hh >= 1) & (hh <= _H) & (ww >= 1) & (ww <= _W))
    mask = mask.astype(jnp.float32).reshape(_P, 1)

    sc1_rows, sh1_rows, w1_rows = [], [], []
    col_off = (c_in, c_in + growth, 0, growth)
    for l in range(4):
        scale, shift = _bn_fold(bn1s[l])
        c_l = scale.shape[0]
        w1 = jnp.transpose(w1s[l][:, :, 0, 0])
        if c_l <= _LANE:
            sc1_rows.append(_pad_lanes(scale, _LANE))
            sh1_rows.append(_pad_lanes(shift, _LANE))
            w1_rows.append(jnp.pad(w1, ((0, _LANE - c_l), (0, 0))))
        else:
            sc1_rows.append(scale[:_LANE].reshape(1, _LANE))
            sh1_rows.append(shift[:_LANE].reshape(1, _LANE))
            w1_rows.append(w1[:_LANE])
            extra = c_l - _LANE
            sc1_b = _pad_lanes(scale[_LANE:], _LANE)
            sh1_b = _pad_lanes(shift[_LANE:], _LANE)
            w1_b = jnp.pad(w1[_LANE:], ((0, _LANE - extra), (0, 0)))
    sc1 = jnp.stack(sc1_rows + [sc1_b])
    sh1 = jnp.stack(sh1_rows + [sh1_b])
    w1p = jnp.stack(w1_rows + [w1_b])

    sc2_rows, sh2_rows, w2_rows = [], [], []
    for l in range(4):
        scale, shift = _bn_fold(bn2s[l])
        sc2_rows.append(scale.reshape(1, _LANE))
        sh2_rows.append(shift.reshape(1, _LANE))
        w9 = jnp.transpose(w2s[l], (2, 3, 1, 0)).reshape(9, c_mid, growth)
        w9 = jnp.pad(w9, ((0, 0), (0, 0),
                          (col_off[l], _LANE - growth - col_off[l])))
        w2_rows.append(w9)
    sc2 = jnp.stack(sc2_rows)
    sh2 = jnp.stack(sh2_rows)
    w2p = jnp.concatenate(w2_rows)

    c_total = c_in + 4 * growth
    out3, spk = pl.pallas_call(
        _block_kernel,
        grid=(b // 2,),
        in_specs=[
            pl.BlockSpec((2, c_in, _H * _W), lambda i: (i, 0, 0)),
            pl.BlockSpec((_P, 1), lambda i: (0, 0)),
            pl.BlockSpec((5, 1, _LANE), lambda i: (0, 0, 0)),
            pl.BlockSpec((5, 1, _LANE), lambda i: (0, 0, 0)),
            pl.BlockSpec((5, _LANE, _LANE), lambda i: (0, 0, 0)),
            pl.BlockSpec((4, 1, _LANE), lambda i: (0, 0, 0)),
            pl.BlockSpec((4, 1, _LANE), lambda i: (0, 0, 0)),
            pl.BlockSpec((36, _LANE, _LANE), lambda i: (0, 0, 0)),
        ],
        out_specs=(
            pl.BlockSpec((2, c_total, _H * _W), lambda i: (i, 0, 0)),
            pl.BlockSpec((2, 1, _LANE), lambda i: (i, 0, 0)),
        ),
        out_shape=(
            jax.ShapeDtypeStruct((b, c_total, _H * _W), jnp.float32),
            jax.ShapeDtypeStruct((b, 1, _LANE), jnp.float32),
        ),
        scratch_shapes=[pltpu.VMEM((_P + 2 * _M, _LANE), jnp.float32),
                        pltpu.VMEM((_P + 2 * _M, _LANE), jnp.float32)],
        compiler_params=pltpu.CompilerParams(
            dimension_semantics=("parallel",),
            vmem_limit_bytes=_VMEM_LIMIT),
    )(x_in, mask, sc1, sh1, w1p, sc2, sh2, w2p)

    out = out3.reshape(b, c_total, _H, _W)

    c_spikes = jnp.sum(spk)
    n2_total = jnp.float32(4 * b * _H * _W * c_mid)
    c_spike_n = c_spikes + n2_total
    return out, c_spikes, c_spike_n


def kernel(x, l0_bn1, l0_w1, l0_bn2, l0_w2, l1_bn1, l1_w1, l1_bn2, l1_w2,
           l2_bn1, l2_w1, l2_bn2, l2_w2, l3_bn1, l3_w1, l3_bn2, l3_w2):
    bn1s = (l0_bn1, l1_bn1, l2_bn1, l3_bn1)
    w1s = (l0_w1, l1_w1, l2_w1, l3_w1)
    bn2s = (l0_bn2, l1_bn2, l2_bn2, l3_bn2)
    w2s = (l0_w2, l1_w2, l2_w2, l3_w2)
    return _forward(x, bn1s, w1s, bn2s, w2s)


# speedup vs baseline: 1.0681x; 1.0681x over previous
import functools

import jax
import jax.numpy as jnp
from jax.experimental import pallas as pl
from jax.experimental.pallas import tpu as pltpu

_D_VALS = (1.5, 0.75, 0.3725, 0.18625, 0.093125)
_BN_EPS = 1e-5
_LANE = 128
_H = 32
_W = 32
_HP = _H + 2
_WP = _W + 2
_P = _HP * _WP
_M = 40
_VMEM_LIMIT = 64 * 1024 * 1024


_R = tuple(1.0 / d for d in _D_VALS)
_CNT_W = (_R[0], _R[1] - _R[0], _R[2] - _R[1], _R[3] - _R[2],
          _R[4] - _R[3], -_R[4])


def _fs_code(act, spk_vec):
    c = act
    spk_vec = spk_vec + _CNT_W[0] * jnp.sum(act, axis=0, keepdims=True)
    for d, w in zip(_D_VALS, _CNT_W[1:]):
        c = jnp.where(c > d, c - d, c)
        spk_vec = spk_vec + w * jnp.sum(c, axis=0, keepdims=True)
    return act - c, spk_vec


def _to_padded_slab(x_img):
    x_wide = jnp.concatenate(
        [x_img, jnp.zeros((_LANE - x_img.shape[0], _H * _W), jnp.float32)],
        axis=0)
    xt = jnp.transpose(x_wide, (1, 0))
    z_ring = jnp.zeros((_WP + 1, _LANE), jnp.float32)
    z_edge = jnp.zeros((2, _LANE), jnp.float32)
    pieces = [z_ring]
    for r in range(_H):
        pieces.append(xt[r * _W:(r + 1) * _W])
        pieces.append(z_edge if r < _H - 1 else z_ring)
    return jnp.concatenate(pieces, axis=0)


def _interior_t(slab):
    rows = [slab[34 * r + 35:34 * r + 67] for r in range(_H)]
    return jnp.transpose(jnp.concatenate(rows, axis=0), (1, 0))


def _block_kernel(x_ref, mask_ref, sc1_ref, sh1_ref, w1_ref, sc2_ref,
                  sh2_ref, w2_ref, out_ref, spk_ref, zbuf0, zbuf1):
    mask = mask_ref[...]
    zbufs = (zbuf0, zbuf1)
    slab0 = [_to_padded_slab(x_ref[0]), _to_padded_slab(x_ref[1])]
    slab1 = [jnp.zeros((_P, _LANE), jnp.float32) for _ in range(2)]
    cnt = [jnp.zeros((1, _LANE), jnp.float32) for _ in range(2)]

    for zb in zbufs:
        zb[pl.ds(0, _M), :] = jnp.zeros((_M, _LANE), jnp.float32)
        zb[pl.ds(_M + _P, _M), :] = jnp.zeros((_M, _LANE), jnp.float32)

    for l in range(4):
        y = [None, None]
        for g in range(2):
            act = jnp.maximum(slab0[g] * sc1_ref[l] + sh1_ref[l], 0.0) * mask
            zw, cnt[g] = _fs_code(act, cnt[g])
            y[g] = jnp.dot(zw, w1_ref[l],
                           preferred_element_type=jnp.float32)
            if l == 3:
                act_b = jnp.maximum(slab1[g] * sc1_ref[4] + sh1_ref[4],
                                    0.0) * mask
                zw_b, cnt[g] = _fs_code(act_b, cnt[g])
                y[g] = y[g] + jnp.dot(zw_b, w1_ref[4],
                                      preferred_element_type=jnp.float32)

        for g in range(2):
            act2 = jnp.maximum(y[g] * sc2_ref[l] + sh2_ref[l], 0.0) * mask
            zw2, cnt[g] = _fs_code(act2, cnt[g])
            zbufs[g][pl.ds(_M, _P), :] = zw2
            acc = jnp.zeros((_P, _LANE), jnp.float32)
            for t in range(9):
                ky, kx = t // 3, t % 3
                off = _M + (ky - 1) * _WP + (kx - 1)
                acc = acc + jnp.dot(zbufs[g][pl.ds(off, _P), :],
                                    w2_ref[9 * l + t],
                                    preferred_element_type=jnp.float32)
            if l < 2:
                slab0[g] = slab0[g] + acc
            else:
                slab1[g] = slab1[g] + acc

    for g in range(2):
        t0 = _interior_t(slab0[g])
        t1 = _interior_t(slab1[g])
        out_ref[g, 0:_LANE, :] = t0
        out_ref[g, _LANE:_LANE + 64, :] = t1[0:64, :]
        spk_ref[g] = cnt[g]


def _bn_fold(bn):
    gamma, beta, mean, var = bn[0], bn[1], bn[2], bn[3]
    scale = gamma / jnp.sqrt(var + _BN_EPS)
    return scale, beta - mean * scale


def _pad_lanes(v, width):
    return jnp.pad(v, (0, width - v.shape[0])).reshape(1, width)


@functools.partial(jax.jit, static_argnames=())
def _forward(x, bn1s, w1s, bn2s, w2s):
    b, c_in = x.shape[0], x.shape[1]
    growth = w2s[0].shape[0]
    c_mid = w2s[0].shape[1]

    x_in = x.reshape(b, c_in, _H * _W)

    hh = jnp.arange(_HP).reshape(_HP, 1)
    ww = jnp.arange(_WP).reshape(1, _WP)
    mask = ((hh >= 1) & (hh <= _H) & (ww >= 1) & (ww <= _W))
    mask = mask.astype(jnp.float32).reshape(_P, 1)

    sc1_rows, sh1_rows, w1_rows = [], [], []
    col_off = (c_in, c_in + growth, 0, growth)
    for l in range(4):
        scale, shift = _bn_fold(bn1s[l])
        c_l = scale.shape[0]
        w1 = jnp.transpose(w1s[l][:, :, 0, 0])
        if c_l <= _LANE:
            sc1_rows.append(_pad_lanes(scale, _LANE))
            sh1_rows.append(_pad_lanes(shift, _LANE))
            w1_rows.append(jnp.pad(w1, ((0, _LANE - c_l), (0, 0))))
        else:
            sc1_rows.append(scale[:_LANE].reshape(1, _LANE))
            sh1_rows.append(shift[:_LANE].reshape(1, _LANE))
            w1_rows.append(w1[:_LANE])
            extra = c_l - _LANE
            sc1_b = _pad_lanes(scale[_LANE:], _LANE)
            sh1_b = _pad_lanes(shift[_LANE:], _LANE)
            w1_b = jnp.pad(w1[_LANE:], ((0, _LANE - extra), (0, 0)))
    sc1 = jnp.stack(sc1_rows + [sc1_b])
    sh1 = jnp.stack(sh1_rows + [sh1_b])
    w1p = jnp.stack(w1_rows + [w1_b])

    sc2_rows, sh2_rows, w2_rows = [], [], []
    for l in range(4):
        scale, shift = _bn_fold(bn2s[l])
        sc2_rows.append(scale.reshape(1, _LANE))
        sh2_rows.append(shift.reshape(1, _LANE))
        w9 = jnp.transpose(w2s[l], (2, 3, 1, 0)).reshape(9, c_mid, growth)
        w9 = jnp.pad(w9, ((0, 0), (0, 0),
                          (col_off[l], _LANE - growth - col_off[l])))
        w2_rows.append(w9)
    sc2 = jnp.stack(sc2_rows)
    sh2 = jnp.stack(sh2_rows)
    w2p = jnp.concatenate(w2_rows)

    c_total = c_in + 4 * growth
    out3, spk = pl.pallas_call(
        _block_kernel,
        grid=(b // 2,),
        in_specs=[
            pl.BlockSpec((2, c_in, _H * _W), lambda i: (i, 0, 0)),
            pl.BlockSpec((_P, 1), lambda i: (0, 0)),
            pl.BlockSpec((5, 1, _LANE), lambda i: (0, 0, 0)),
            pl.BlockSpec((5, 1, _LANE), lambda i: (0, 0, 0)),
            pl.BlockSpec((5, _LANE, _LANE), lambda i: (0, 0, 0)),
            pl.BlockSpec((4, 1, _LANE), lambda i: (0, 0, 0)),
            pl.BlockSpec((4, 1, _LANE), lambda i: (0, 0, 0)),
            pl.BlockSpec((36, _LANE, _LANE), lambda i: (0, 0, 0)),
        ],
        out_specs=(
            pl.BlockSpec((2, c_total, _H * _W), lambda i: (i, 0, 0)),
            pl.BlockSpec((2, 1, _LANE), lambda i: (i, 0, 0)),
        ),
        out_shape=(
            jax.ShapeDtypeStruct((b, c_total, _H * _W), jnp.float32),
            jax.ShapeDtypeStruct((b, 1, _LANE), jnp.float32),
        ),
        scratch_shapes=[pltpu.VMEM((_P + 2 * _M, _LANE), jnp.float32),
                        pltpu.VMEM((_P + 2 * _M, _LANE), jnp.float32)],
        compiler_params=pltpu.CompilerParams(
            dimension_semantics=("parallel",),
            vmem_limit_bytes=_VMEM_LIMIT),
    )(x_in, mask, sc1, sh1, w1p, sc2, sh2, w2p)

    out = out3.reshape(b, c_total, _H, _W)

    c_spikes = jnp.sum(spk)
    n2_total = jnp.float32(4 * b * _H * _W * c_mid)
    c_spike_n = c_spikes + n2_total
    return out, c_spikes, c_spike_n


def kernel(x, l0_bn1, l0_w1, l0_bn2, l0_w2, l1_bn1, l1_w1, l1_bn2, l1_w2,
           l2_bn1, l2_w1, l2_bn2, l2_w2, l3_bn1, l3_w1, l3_bn2, l3_w2):
    bn1s = (l0_bn1, l1_bn1, l2_bn1, l3_bn1)
    w1s = (l0_w1, l1_w1, l2_w1, l3_w1)
    bn2s = (l0_bn2, l1_bn2, l2_bn2, l3_bn2)
    w2s = (l0_w2, l1_w2, l2_w2, l3_w2)
    return _forward(x, bn1s, w1s, bn2s, w2s)


# speedup vs baseline: 1.1090x; 1.0382x over previous
import functools

import jax
import jax.numpy as jnp
from jax.experimental import pallas as pl
from jax.experimental.pallas import tpu as pltpu

_D_VALS = (1.5, 0.75, 0.3725, 0.18625, 0.093125)
_BN_EPS = 1e-5
_LANE = 128
_H = 32
_W = 32
_N = _H * _W
_HP = _H + 2
_WP = _W + 2
_P = _HP * _WP
_M = 40
_VMEM_LIMIT = 64 * 1024 * 1024

_R = tuple(1.0 / d for d in _D_VALS)
_CNT_W = (_R[0], _R[1] - _R[0], _R[2] - _R[1], _R[3] - _R[2],
          _R[4] - _R[3], -_R[4])


def _fs_code(act, spk_vec):
    c = act
    spk_vec = spk_vec + _CNT_W[0] * jnp.sum(act, axis=0, keepdims=True)
    for d, w in zip(_D_VALS, _CNT_W[1:]):
        c = jnp.where(c > d, c - d, c)
        spk_vec = spk_vec + w * jnp.sum(c, axis=0, keepdims=True)
    return act - c, spk_vec


def _interior(acc):
    return jnp.concatenate(
        [acc[34 * r + 35:34 * r + 67] for r in range(_H)], axis=0)


def _block_kernel(x_ref, sc1_ref, sh1_ref, w1_ref, sc2_ref,
                  sh2_ref, w2_ref, out_ref, spk_ref, zbuf0, zbuf1):
    zbufs = (zbuf0, zbuf1)
    slab0 = []
    for g in range(2):
        x_wide = jnp.concatenate(
            [x_ref[g], jnp.zeros((_LANE - x_ref.shape[1], _N), jnp.float32)],
            axis=0)
        slab0.append(jnp.transpose(x_wide, (1, 0)))
    slab1 = [jnp.zeros((_N, _LANE), jnp.float32) for _ in range(2)]
    cnt = [jnp.zeros((1, _LANE), jnp.float32) for _ in range(2)]

    for zb in zbufs:
        zb[...] = jnp.zeros((_M + _P + _M, _LANE), jnp.float32)

    for l in range(4):
        y = [None, None]
        for g in range(2):
            act = jnp.maximum(slab0[g] * sc1_ref[l] + sh1_ref[l], 0.0)
            zw, cnt[g] = _fs_code(act, cnt[g])
            y[g] = jnp.dot(zw, w1_ref[l],
                           preferred_element_type=jnp.float32)
            if l == 3:
                act_b = jnp.maximum(slab1[g] * sc1_ref[4] + sh1_ref[4], 0.0)
                zw_b, cnt[g] = _fs_code(act_b, cnt[g])
                y[g] = y[g] + jnp.dot(zw_b, w1_ref[4],
                                      preferred_element_type=jnp.float32)

        for g in range(2):
            act2 = jnp.maximum(y[g] * sc2_ref[l] + sh2_ref[l], 0.0)
            zw2, cnt[g] = _fs_code(act2, cnt[g])
            for r in range(_H):
                zbufs[g][pl.ds(_M + 35 + 34 * r, _W), :] = \
                    zw2[r * _W:(r + 1) * _W]
            acc = jnp.zeros((_P, _LANE), jnp.float32)
            for t in range(9):
                ky, kx = t // 3, t % 3
                off = _M + (ky - 1) * _WP + (kx - 1)
                acc = acc + jnp.dot(zbufs[g][pl.ds(off, _P), :],
                                    w2_ref[9 * l + t],
                                    preferred_element_type=jnp.float32)
            if l < 2:
                slab0[g] = slab0[g] + _interior(acc)
            else:
                slab1[g] = slab1[g] + _interior(acc)

    for g in range(2):
        t0 = jnp.transpose(slab0[g], (1, 0))
        t1 = jnp.transpose(slab1[g], (1, 0))
        out_ref[g, 0:_LANE, :] = t0
        out_ref[g, _LANE:_LANE + 64, :] = t1[0:64, :]
        spk_ref[g] = cnt[g]


def _bn_fold(bn):
    gamma, beta, mean, var = bn[0], bn[1], bn[2], bn[3]
    scale = gamma / jnp.sqrt(var + _BN_EPS)
    return scale, beta - mean * scale


def _pad_lanes(v, width):
    return jnp.pad(v, (0, width - v.shape[0])).reshape(1, width)


@functools.partial(jax.jit, static_argnames=())
def _forward(x, bn1s, w1s, bn2s, w2s):
    b, c_in = x.shape[0], x.shape[1]
    growth = w2s[0].shape[0]
    c_mid = w2s[0].shape[1]

    x_in = x.reshape(b, c_in, _N)

    sc1_rows, sh1_rows, w1_rows = [], [], []
    col_off = (c_in, c_in + growth, 0, growth)
    for l in range(4):
        scale, shift = _bn_fold(bn1s[l])
        c_l = scale.shape[0]
        w1 = jnp.transpose(w1s[l][:, :, 0, 0])
        if c_l <= _LANE:
            sc1_rows.append(_pad_lanes(scale, _LANE))
            sh1_rows.append(_pad_lanes(shift, _LANE))
            w1_rows.append(jnp.pad(w1, ((0, _LANE - c_l), (0, 0))))
        else:
            sc1_rows.append(scale[:_LANE].reshape(1, _LANE))
            sh1_rows.append(shift[:_LANE].reshape(1, _LANE))
            w1_rows.append(w1[:_LANE])
            extra = c_l - _LANE
            sc1_b = _pad_lanes(scale[_LANE:], _LANE)
            sh1_b = _pad_lanes(shift[_LANE:], _LANE)
            w1_b = jnp.pad(w1[_LANE:], ((0, _LANE - extra), (0, 0)))
    sc1 = jnp.stack(sc1_rows + [sc1_b])
    sh1 = jnp.stack(sh1_rows + [sh1_b])
    w1p = jnp.stack(w1_rows + [w1_b])

    sc2_rows, sh2_rows, w2_rows = [], [], []
    for l in range(4):
        scale, shift = _bn_fold(bn2s[l])
        sc2_rows.append(scale.reshape(1, _LANE))
        sh2_rows.append(shift.reshape(1, _LANE))
        w9 = jnp.transpose(w2s[l], (2, 3, 1, 0)).reshape(9, c_mid, growth)
        w9 = jnp.pad(w9, ((0, 0), (0, 0),
                          (col_off[l], _LANE - growth - col_off[l])))
        w2_rows.append(w9)
    sc2 = jnp.stack(sc2_rows)
    sh2 = jnp.stack(sh2_rows)
    w2p = jnp.concatenate(w2_rows)

    c_total = c_in + 4 * growth
    out3, spk = pl.pallas_call(
        _block_kernel,
        grid=(b // 2,),
        in_specs=[
            pl.BlockSpec((2, c_in, _N), lambda i: (i, 0, 0)),
            pl.BlockSpec((5, 1, _LANE), lambda i: (0, 0, 0)),
            pl.BlockSpec((5, 1, _LANE), lambda i: (0, 0, 0)),
            pl.BlockSpec((5, _LANE, _LANE), lambda i: (0, 0, 0)),
            pl.BlockSpec((4, 1, _LANE), lambda i: (0, 0, 0)),
            pl.BlockSpec((4, 1, _LANE), lambda i: (0, 0, 0)),
            pl.BlockSpec((36, _LANE, _LANE), lambda i: (0, 0, 0)),
        ],
        out_specs=(
            pl.BlockSpec((2, c_total, _N), lambda i: (i, 0, 0)),
            pl.BlockSpec((2, 1, _LANE), lambda i: (i, 0, 0)),
        ),
        out_shape=(
            jax.ShapeDtypeStruct((b, c_total, _N), jnp.float32),
            jax.ShapeDtypeStruct((b, 1, _LANE), jnp.float32),
        ),
        scratch_shapes=[pltpu.VMEM((_P + 2 * _M, _LANE), jnp.float32),
                        pltpu.VMEM((_P + 2 * _M, _LANE), jnp.float32)],
        compiler_params=pltpu.CompilerParams(
            dimension_semantics=("parallel",),
            vmem_limit_bytes=_VMEM_LIMIT),
    )(x_in, sc1, sh1, w1p, sc2, sh2, w2p)

    out = out3.reshape(b, c_total, _H, _W)
    c_spikes = jnp.sum(spk)
    n2_total = jnp.float32(4 * b * _N * c_mid)
    c_spike_n = c_spikes + n2_total
    return out, c_spikes, c_spike_n


def kernel(x, l0_bn1, l0_w1, l0_bn2, l0_w2, l1_bn1, l1_w1, l1_bn2, l1_w2,
           l2_bn1, l2_w1, l2_bn2, l2_w2, l3_bn1, l3_w1, l3_bn2, l3_w2):
    bn1s = (l0_bn1, l1_bn1, l2_bn1, l3_bn1)
    w1s = (l0_w1, l1_w1, l2_w1, l3_w1)
    bn2s = (l0_bn2, l1_bn2, l2_bn2, l3_bn2)
    w2s = (l0_w2, l1_w2, l2_w2, l3_w2)
    return _forward(x, bn1s, w1s, bn2s, w2s)


# speedup vs baseline: 1.1637x; 1.0493x over previous
import functools

import jax
import jax.numpy as jnp
from jax.experimental import pallas as pl
from jax.experimental.pallas import tpu as pltpu

_D_VALS = (1.5, 0.75, 0.3725, 0.18625, 0.093125)
_BN_EPS = 1e-5
_LANE = 128
_H = 32
_W = 32
_N = _H * _W
_HP = _H + 2
_WP = _W + 2
_P = _HP * _WP
_M = 40
_VMEM_LIMIT = 96 * 1024 * 1024
_G = 4

_R = tuple(1.0 / d for d in _D_VALS)
_CNT_W = (_R[0], _R[1] - _R[0], _R[2] - _R[1], _R[3] - _R[2],
          _R[4] - _R[3], -_R[4])


def _fs_code(act, spk_vec):
    c = act
    spk_vec = spk_vec + _CNT_W[0] * jnp.sum(act, axis=0, keepdims=True)
    for d, w in zip(_D_VALS, _CNT_W[1:]):
        c = jnp.where(c > d, c - d, c)
        spk_vec = spk_vec + w * jnp.sum(c, axis=0, keepdims=True)
    return act - c, spk_vec


def _interior(acc):
    return jnp.concatenate(
        [acc[34 * r + 35:34 * r + 67] for r in range(_H)], axis=0)


def _block_kernel(x_ref, sc1_ref, sh1_ref, w1_ref, sc2_ref,
                  sh2_ref, w2_ref, out_ref, spk_ref, *zbufs):
    slab0 = []
    for g in range(_G):
        x_wide = jnp.concatenate(
            [x_ref[g], jnp.zeros((_LANE - x_ref.shape[1], _N), jnp.float32)],
            axis=0)
        slab0.append(jnp.transpose(x_wide, (1, 0)))
    slab1 = [jnp.zeros((_N, _LANE), jnp.float32) for _ in range(_G)]
    cnt = [jnp.zeros((1, _LANE), jnp.float32) for _ in range(_G)]

    for zb in zbufs:
        zb[...] = jnp.zeros((_M + _P + _M, _LANE), jnp.float32)

    for l in range(4):
        y = [None] * _G
        for g in range(_G):
            act = jnp.maximum(slab0[g] * sc1_ref[l] + sh1_ref[l], 0.0)
            zw, cnt[g] = _fs_code(act, cnt[g])
            y[g] = jnp.dot(zw, w1_ref[l],
                           preferred_element_type=jnp.float32)
            if l == 3:
                act_b = jnp.maximum(slab1[g] * sc1_ref[4] + sh1_ref[4], 0.0)
                zw_b, cnt[g] = _fs_code(act_b, cnt[g])
                y[g] = y[g] + jnp.dot(zw_b, w1_ref[4],
                                      preferred_element_type=jnp.float32)

        for g in range(_G):
            act2 = jnp.maximum(y[g] * sc2_ref[l] + sh2_ref[l], 0.0)
            zw2, cnt[g] = _fs_code(act2, cnt[g])
            for r in range(_H):
                zbufs[g][pl.ds(_M + 35 + 34 * r, _W), :] = \
                    zw2[r * _W:(r + 1) * _W]
            acc = jnp.zeros((_P, _LANE), jnp.float32)
            for t in range(9):
                ky, kx = t // 3, t % 3
                off = _M + (ky - 1) * _WP + (kx - 1)
                acc = acc + jnp.dot(zbufs[g][pl.ds(off, _P), :],
                                    w2_ref[9 * l + t],
                                    preferred_element_type=jnp.float32)
            if l < 2:
                slab0[g] = slab0[g] + _interior(acc)
            else:
                slab1[g] = slab1[g] + _interior(acc)

    for g in range(_G):
        t0 = jnp.transpose(slab0[g], (1, 0))
        t1 = jnp.transpose(slab1[g], (1, 0))
        out_ref[g, 0:_LANE, :] = t0
        out_ref[g, _LANE:_LANE + 64, :] = t1[0:64, :]
        spk_ref[g] = cnt[g]


def _bn_fold(bn):
    gamma, beta, mean, var = bn[0], bn[1], bn[2], bn[3]
    scale = gamma / jnp.sqrt(var + _BN_EPS)
    return scale, beta - mean * scale


def _pad_lanes(v, width):
    return jnp.pad(v, (0, width - v.shape[0])).reshape(1, width)


@functools.partial(jax.jit, static_argnames=())
def _forward(x, bn1s, w1s, bn2s, w2s):
    b, c_in = x.shape[0], x.shape[1]
    growth = w2s[0].shape[0]
    c_mid = w2s[0].shape[1]

    x_in = x.reshape(b, c_in, _N)

    sc1_rows, sh1_rows, w1_rows = [], [], []
    col_off = (c_in, c_in + growth, 0, growth)
    for l in range(4):
        scale, shift = _bn_fold(bn1s[l])
        c_l = scale.shape[0]
        w1 = jnp.transpose(w1s[l][:, :, 0, 0])
        if c_l <= _LANE:
            sc1_rows.append(_pad_lanes(scale, _LANE))
            sh1_rows.append(_pad_lanes(shift, _LANE))
            w1_rows.append(jnp.pad(w1, ((0, _LANE - c_l), (0, 0))))
        else:
            sc1_rows.append(scale[:_LANE].reshape(1, _LANE))
            sh1_rows.append(shift[:_LANE].reshape(1, _LANE))
            w1_rows.append(w1[:_LANE])
            extra = c_l - _LANE
            sc1_b = _pad_lanes(scale[_LANE:], _LANE)
            sh1_b = _pad_lanes(shift[_LANE:], _LANE)
            w1_b = jnp.pad(w1[_LANE:], ((0, _LANE - extra), (0, 0)))
    sc1 = jnp.stack(sc1_rows + [sc1_b])
    sh1 = jnp.stack(sh1_rows + [sh1_b])
    w1p = jnp.stack(w1_rows + [w1_b])

    sc2_rows, sh2_rows, w2_rows = [], [], []
    for l in range(4):
        scale, shift = _bn_fold(bn2s[l])
        sc2_rows.append(scale.reshape(1, _LANE))
        sh2_rows.append(shift.reshape(1, _LANE))
        w9 = jnp.transpose(w2s[l], (2, 3, 1, 0)).reshape(9, c_mid, growth)
        w9 = jnp.pad(w9, ((0, 0), (0, 0),
                          (col_off[l], _LANE - growth - col_off[l])))
        w2_rows.append(w9)
    sc2 = jnp.stack(sc2_rows)
    sh2 = jnp.stack(sh2_rows)
    w2p = jnp.concatenate(w2_rows)

    c_total = c_in + 4 * growth
    out3, spk = pl.pallas_call(
        _block_kernel,
        grid=(b // _G,),
        in_specs=[
            pl.BlockSpec((_G, c_in, _N), lambda i: (i, 0, 0)),
            pl.BlockSpec((5, 1, _LANE), lambda i: (0, 0, 0)),
            pl.BlockSpec((5, 1, _LANE), lambda i: (0, 0, 0)),
            pl.BlockSpec((5, _LANE, _LANE), lambda i: (0, 0, 0)),
            pl.BlockSpec((4, 1, _LANE), lambda i: (0, 0, 0)),
            pl.BlockSpec((4, 1, _LANE), lambda i: (0, 0, 0)),
            pl.BlockSpec((36, _LANE, _LANE), lambda i: (0, 0, 0)),
        ],
        out_specs=(
            pl.BlockSpec((_G, c_total, _N), lambda i: (i, 0, 0)),
            pl.BlockSpec((_G, 1, _LANE), lambda i: (i, 0, 0)),
        ),
        out_shape=(
            jax.ShapeDtypeStruct((b, c_total, _N), jnp.float32),
            jax.ShapeDtypeStruct((b, 1, _LANE), jnp.float32),
        ),
        scratch_shapes=[pltpu.VMEM((_P + 2 * _M, _LANE), jnp.float32)
                        for _ in range(_G)],
        compiler_params=pltpu.CompilerParams(
            dimension_semantics=("parallel",),
            vmem_limit_bytes=_VMEM_LIMIT),
    )(x_in, sc1, sh1, w1p, sc2, sh2, w2p)

    out = out3.reshape(b, c_total, _H, _W)
    c_spikes = jnp.sum(spk)
    n2_total = jnp.float32(4 * b * _N * c_mid)
    c_spike_n = c_spikes + n2_total
    return out, c_spikes, c_spike_n


def kernel(x, l0_bn1, l0_w1, l0_bn2, l0_w2, l1_bn1, l1_w1, l1_bn2, l1_w2,
           l2_bn1, l2_w1, l2_bn2, l2_w2, l3_bn1, l3_w1, l3_bn2, l3_w2):
    bn1s = (l0_bn1, l1_bn1, l2_bn1, l3_bn1)
    w1s = (l0_w1, l1_w1, l2_w1, l3_w1)
    bn2s = (l0_bn2, l1_bn2, l2_bn2, l3_bn2)
    w2s = (l0_w2, l1_w2, l2_w2, l3_w2)
    return _forward(x, bn1s, w1s, bn2s, w2s)
